# baseline (speedup 1.0000x reference)
import jax
import jax.numpy as jnp
from jax.experimental import pallas as pl
from jax.experimental.pallas import tpu as pltpu

DIM = 256
N_Q = 8
BINS = 1024
N_TOK = 2048
TOK_BLOCK = 256


def _dot(a, b, dims):
    return jax.lax.dot_general(a, b, (dims, ((), ())),
                               preferred_element_type=jnp.float32)


def _rvq_body(h_ref, hi_ref, mid_ref, lo_ref, codes_ref, quant_ref,
              r_ref, cn_ref):
    jblk = pl.program_id(0)
    lvl = pl.program_id(1)

    c_hi = hi_ref[0]
    c_mid = mid_ref[0]
    c_lo = lo_ref[0]

    cb_f32 = (c_hi.astype(jnp.float32) + c_mid.astype(jnp.float32)
              ) + c_lo.astype(jnp.float32)
    cbsq = cb_f32 * cb_f32
    sq_hi = cbsq.astype(jnp.bfloat16)
    sq_lo = (cbsq - sq_hi.astype(jnp.float32)).astype(jnp.bfloat16)
    ones_row = jnp.ones((1, DIM), jnp.bfloat16)
    cnorm = (_dot(ones_row, sq_hi, ((1,), (1,)))
             + _dot(ones_row, sq_lo, ((1,), (1,))))

    @pl.when(lvl == 0)
    def _():
        r_ref[...] = h_ref[...]
        quant_ref[...] = jnp.zeros_like(quant_ref)

    r = r_ref[...]
    r_hi = r.astype(jnp.bfloat16)
    r_lo = (r - r_hi.astype(jnp.float32)).astype(jnp.bfloat16)
    lane = jax.lax.broadcasted_iota(jnp.int32, (r.shape[0], BINS), 1)
    rc = (_dot(r_hi, c_hi, ((1,), (1,)))
          + _dot(r_hi, c_mid, ((1,), (1,)))
          + _dot(r_lo, c_hi, ((1,), (1,))))
    scores = cnorm - 2.0 * rc
    m1 = jnp.min(scores, axis=1, keepdims=True)
    i1 = jnp.min(jnp.where(scores == m1, lane, BINS), axis=1, keepdims=True)
    masked = jnp.where(lane == i1, jnp.inf, scores)
    m2 = jnp.min(masked, axis=1, keepdims=True)
    i2 = jnp.min(jnp.where(masked == m2, lane, BINS), axis=1, keepdims=True)
    t = r.shape[0]
    oh1 = (lane == i1).astype(jnp.bfloat16)
    oh2 = (lane == i2).astype(jnp.bfloat16)
    c1 = (_dot(oh1, c_hi, ((1,), (0,)))
          + _dot(oh1, c_mid, ((1,), (0,)))
          + _dot(oh1, c_lo, ((1,), (0,))))
    c2 = (_dot(oh2, c_hi, ((1,), (0,)))
          + _dot(oh2, c_mid, ((1,), (0,)))
          + _dot(oh2, c_lo, ((1,), (0,))))
    d1 = jnp.sum((r - c1) ** 2, axis=1, keepdims=True)
    d2 = jnp.sum((r - c2) ** 2, axis=1, keepdims=True)
    pick2 = (d2 < d1) | ((d2 == d1) & (i2 < i1))
    chosen = jnp.where(pick2, c2, c1)
    idx_col = jnp.where(pick2, i2, i1)
    digits = jnp.concatenate(
        [idx_col >> 8, idx_col & 255], axis=1).astype(jnp.bfloat16)
    eye = (jax.lax.broadcasted_iota(jnp.int32, (t, t), 0) ==
           jax.lax.broadcasted_iota(jnp.int32, (t, t), 1)).astype(jnp.bfloat16)
    ab_row = _dot(digits, eye, ((0,), (0,)))
    idx_row = ab_row[:1] * 256.0 + ab_row[1:]
    codes_ref[0] = idx_row.astype(jnp.int32)
    quant_ref[...] += chosen
    r_ref[...] = r - chosen


def kernel(hidden_states, codebooks):
    cb_hi = jax.lax.optimization_barrier(codebooks.astype(jnp.bfloat16))
    res1 = codebooks - cb_hi.astype(jnp.float32)
    cb_mid = jax.lax.optimization_barrier(res1.astype(jnp.bfloat16))
    cb_lo = (res1 - cb_mid.astype(jnp.float32)).astype(jnp.bfloat16)

    grid = (N_TOK // TOK_BLOCK, N_Q)
    codes3, quantized = pl.pallas_call(
        _rvq_body,
        grid=grid,
        in_specs=[
            pl.BlockSpec((TOK_BLOCK, DIM), lambda j, i: (j, 0)),
            pl.BlockSpec((1, BINS, DIM), lambda j, i: (i, 0, 0)),
            pl.BlockSpec((1, BINS, DIM), lambda j, i: (i, 0, 0)),
            pl.BlockSpec((1, BINS, DIM), lambda j, i: (i, 0, 0)),
        ],
        out_specs=[
            pl.BlockSpec((1, 1, TOK_BLOCK), lambda j, i: (i, 0, j)),
            pl.BlockSpec((TOK_BLOCK, DIM), lambda j, i: (j, 0)),
        ],
        out_shape=[
            jax.ShapeDtypeStruct((N_Q, 1, N_TOK), jnp.int32),
            jax.ShapeDtypeStruct((N_TOK, DIM), jnp.float32),
        ],
        scratch_shapes=[
            pltpu.VMEM((TOK_BLOCK, DIM), jnp.float32),
            pltpu.VMEM((N_Q, 1, BINS), jnp.float32),
        ],
    )(hidden_states, cb_hi, cb_mid, cb_lo)
    return codes3.reshape(N_Q, N_TOK), quantized


# speedup vs baseline: 1.1659x; 1.1659x over previous
import jax
import jax.numpy as jnp
from jax.experimental import pallas as pl
from jax.experimental.pallas import tpu as pltpu

DIM = 256
N_Q = 8
BINS = 1024
N_TOK = 2048
TOK_BLOCK = 256


def _dot(a, b, dims):
    return jax.lax.dot_general(a, b, (dims, ((), ())),
                               preferred_element_type=jnp.float32)


def _rvq_body(h_ref, hi_ref, mid_ref, lo_ref, codes_ref, quant_ref,
              r_ref, cn_ref):
    jblk = pl.program_id(0)
    lvl = pl.program_id(1)

    c_hi = hi_ref[0]
    c_mid = mid_ref[0]
    c_lo = lo_ref[0]

    @pl.when(jblk == 0)
    def _():
        cb_f32 = (c_hi.astype(jnp.float32) + c_mid.astype(jnp.float32)
                  ) + c_lo.astype(jnp.float32)
        cbsq = cb_f32 * cb_f32
        sq_hi = cbsq.astype(jnp.bfloat16)
        sq_lo = (cbsq - sq_hi.astype(jnp.float32)).astype(jnp.bfloat16)
        ones_row = jnp.ones((1, DIM), jnp.bfloat16)
        cn_ref[lvl] = (_dot(ones_row, sq_hi, ((1,), (1,)))
                       + _dot(ones_row, sq_lo, ((1,), (1,))))

    cnorm = cn_ref[lvl]

    @pl.when(lvl == 0)
    def _():
        r_ref[...] = h_ref[...]
        quant_ref[...] = jnp.zeros_like(quant_ref)

    r = r_ref[...]
    r_hi = r.astype(jnp.bfloat16)
    r_lo = (r - r_hi.astype(jnp.float32)).astype(jnp.bfloat16)
    lane = jax.lax.broadcasted_iota(jnp.int32, (r.shape[0], BINS), 1)
    rc = (_dot(r_hi, c_hi, ((1,), (1,)))
          + _dot(r_hi, c_mid, ((1,), (1,)))
          + _dot(r_lo, c_hi, ((1,), (1,))))
    scores = cnorm - 2.0 * rc
    m1 = jnp.min(scores, axis=1, keepdims=True)
    i1 = jnp.min(jnp.where(scores == m1, lane, BINS), axis=1, keepdims=True)
    masked = jnp.where(lane == i1, jnp.inf, scores)
    m2 = jnp.min(masked, axis=1, keepdims=True)
    i2 = jnp.min(jnp.where(masked == m2, lane, BINS), axis=1, keepdims=True)
    t = r.shape[0]
    oh = jnp.concatenate([(lane == i1).astype(jnp.float32),
                          (lane == i2).astype(jnp.float32)],
                         axis=0).astype(jnp.bfloat16)
    c12 = (_dot(oh, c_hi, ((1,), (0,)))
           + _dot(oh, c_mid, ((1,), (0,)))
           + _dot(oh, c_lo, ((1,), (0,))))
    c1 = c12[:t]
    c2 = c12[t:]
    d1 = jnp.sum((r - c1) ** 2, axis=1, keepdims=True)
    d2 = jnp.sum((r - c2) ** 2, axis=1, keepdims=True)
    pick2 = (d2 < d1) | ((d2 == d1) & (i2 < i1))
    chosen = jnp.where(pick2, c2, c1)
    idx_col = jnp.where(pick2, i2, i1)
    digits = jnp.concatenate(
        [idx_col >> 8, idx_col & 255], axis=1).astype(jnp.bfloat16)
    eye = (jax.lax.broadcasted_iota(jnp.int32, (t, t), 0) ==
           jax.lax.broadcasted_iota(jnp.int32, (t, t), 1)).astype(jnp.bfloat16)
    ab_row = _dot(digits, eye, ((0,), (0,)))
    idx_row = ab_row[:1] * 256.0 + ab_row[1:]
    codes_ref[0] = idx_row.astype(jnp.int32)
    quant_ref[...] += chosen
    r_ref[...] = r - chosen


def kernel(hidden_states, codebooks):
    cb_hi = jax.lax.optimization_barrier(codebooks.astype(jnp.bfloat16))
    res1 = codebooks - cb_hi.astype(jnp.float32)
    cb_mid = jax.lax.optimization_barrier(res1.astype(jnp.bfloat16))
    cb_lo = (res1 - cb_mid.astype(jnp.float32)).astype(jnp.bfloat16)

    grid = (N_TOK // TOK_BLOCK, N_Q)
    codes3, quantized = pl.pallas_call(
        _rvq_body,
        grid=grid,
        in_specs=[
            pl.BlockSpec((TOK_BLOCK, DIM), lambda j, i: (j, 0)),
            pl.BlockSpec((1, BINS, DIM), lambda j, i: (i, 0, 0)),
            pl.BlockSpec((1, BINS, DIM), lambda j, i: (i, 0, 0)),
            pl.BlockSpec((1, BINS, DIM), lambda j, i: (i, 0, 0)),
        ],
        out_specs=[
            pl.BlockSpec((1, 1, TOK_BLOCK), lambda j, i: (i, 0, j)),
            pl.BlockSpec((TOK_BLOCK, DIM), lambda j, i: (j, 0)),
        ],
        out_shape=[
            jax.ShapeDtypeStruct((N_Q, 1, N_TOK), jnp.int32),
            jax.ShapeDtypeStruct((N_TOK, DIM), jnp.float32),
        ],
        scratch_shapes=[
            pltpu.VMEM((TOK_BLOCK, DIM), jnp.float32),
            pltpu.VMEM((N_Q, 1, BINS), jnp.float32),
        ],
    )(hidden_states, cb_hi, cb_mid, cb_lo)
    return codes3.reshape(N_Q, N_TOK), quantized


# speedup vs baseline: 1.3435x; 1.1523x over previous
import jax
import jax.numpy as jnp
from jax.experimental import pallas as pl
from jax.experimental.pallas import tpu as pltpu

DIM = 256
N_Q = 8
BINS = 1024
N_TOK = 2048
TOK_BLOCK = 512


def _dot(a, b, dims):
    return jax.lax.dot_general(a, b, (dims, ((), ())),
                               preferred_element_type=jnp.float32)


def _rvq_body(h_ref, hi_ref, mid_ref, lo_ref, codes_ref, quant_ref,
              r_ref, cn_ref):
    jblk = pl.program_id(0)
    lvl = pl.program_id(1)

    c_hi = hi_ref[0]
    c_mid = mid_ref[0]
    c_lo = lo_ref[0]

    @pl.when(jblk == 0)
    def _():
        cb_f32 = (c_hi.astype(jnp.float32) + c_mid.astype(jnp.float32)
                  ) + c_lo.astype(jnp.float32)
        cbsq = cb_f32 * cb_f32
        sq_hi = cbsq.astype(jnp.bfloat16)
        sq_lo = (cbsq - sq_hi.astype(jnp.float32)).astype(jnp.bfloat16)
        ones_row = jnp.ones((1, DIM), jnp.bfloat16)
        cn_ref[lvl] = (_dot(ones_row, sq_hi, ((1,), (1,)))
                       + _dot(ones_row, sq_lo, ((1,), (1,))))

    cnorm = cn_ref[lvl]

    @pl.when(lvl == 0)
    def _():
        r_ref[...] = h_ref[...]
        quant_ref[...] = jnp.zeros_like(quant_ref)

    r = r_ref[...]
    r_hi = r.astype(jnp.bfloat16)
    r_lo = (r - r_hi.astype(jnp.float32)).astype(jnp.bfloat16)
    lane = jax.lax.broadcasted_iota(jnp.int32, (r.shape[0], BINS), 1)
    rc = (_dot(r_hi, c_hi, ((1,), (1,)))
          + _dot(r_hi, c_mid, ((1,), (1,)))
          + _dot(r_lo, c_hi, ((1,), (1,))))
    scores = cnorm - 2.0 * rc
    m1 = jnp.min(scores, axis=1, keepdims=True)
    i1 = jnp.min(jnp.where(scores == m1, lane, BINS), axis=1, keepdims=True)
    masked = jnp.where(lane == i1, jnp.inf, scores)
    m2 = jnp.min(masked, axis=1, keepdims=True)
    i2 = jnp.min(jnp.where(masked == m2, lane, BINS), axis=1, keepdims=True)
    t = r.shape[0]
    oh = jnp.concatenate([(lane == i1).astype(jnp.float32),
                          (lane == i2).astype(jnp.float32)],
                         axis=0).astype(jnp.bfloat16)
    c12 = (_dot(oh, c_hi, ((1,), (0,)))
           + _dot(oh, c_mid, ((1,), (0,)))
           + _dot(oh, c_lo, ((1,), (0,))))
    c1 = c12[:t]
    c2 = c12[t:]
    d1 = jnp.sum((r - c1) ** 2, axis=1, keepdims=True)
    d2 = jnp.sum((r - c2) ** 2, axis=1, keepdims=True)
    pick2 = (d2 < d1) | ((d2 == d1) & (i2 < i1))
    chosen = jnp.where(pick2, c2, c1)
    idx_col = jnp.where(pick2, i2, i1)
    digits = jnp.concatenate(
        [idx_col >> 8, idx_col & 255], axis=1).astype(jnp.bfloat16)
    eye = (jax.lax.broadcasted_iota(jnp.int32, (t, t), 0) ==
           jax.lax.broadcasted_iota(jnp.int32, (t, t), 1)).astype(jnp.bfloat16)
    ab_row = _dot(digits, eye, ((0,), (0,)))
    idx_row = ab_row[:1] * 256.0 + ab_row[1:]
    codes_ref[0] = idx_row.astype(jnp.int32)
    quant_ref[...] += chosen
    r_ref[...] = r - chosen


def kernel(hidden_states, codebooks):
    cb_hi = jax.lax.optimization_barrier(codebooks.astype(jnp.bfloat16))
    res1 = codebooks - cb_hi.astype(jnp.float32)
    cb_mid = jax.lax.optimization_barrier(res1.astype(jnp.bfloat16))
    cb_lo = (res1 - cb_mid.astype(jnp.float32)).astype(jnp.bfloat16)

    grid = (N_TOK // TOK_BLOCK, N_Q)
    codes3, quantized = pl.pallas_call(
        _rvq_body,
        grid=grid,
        in_specs=[
            pl.BlockSpec((TOK_BLOCK, DIM), lambda j, i: (j, 0)),
            pl.BlockSpec((1, BINS, DIM), lambda j, i: (i, 0, 0)),
            pl.BlockSpec((1, BINS, DIM), lambda j, i: (i, 0, 0)),
            pl.BlockSpec((1, BINS, DIM), lambda j, i: (i, 0, 0)),
        ],
        out_specs=[
            pl.BlockSpec((1, 1, TOK_BLOCK), lambda j, i: (i, 0, j)),
            pl.BlockSpec((TOK_BLOCK, DIM), lambda j, i: (j, 0)),
        ],
        out_shape=[
            jax.ShapeDtypeStruct((N_Q, 1, N_TOK), jnp.int32),
            jax.ShapeDtypeStruct((N_TOK, DIM), jnp.float32),
        ],
        scratch_shapes=[
            pltpu.VMEM((TOK_BLOCK, DIM), jnp.float32),
            pltpu.VMEM((N_Q, 1, BINS), jnp.float32),
        ],
    )(hidden_states, cb_hi, cb_mid, cb_lo)
    return codes3.reshape(N_Q, N_TOK), quantized


# speedup vs baseline: 1.4269x; 1.0620x over previous
import jax
import jax.numpy as jnp
from jax.experimental import pallas as pl
from jax.experimental.pallas import tpu as pltpu

DIM = 256
N_Q = 8
BINS = 1024
N_TOK = 2048
TOK_BLOCK = 1024


def _dot(a, b, dims):
    return jax.lax.dot_general(a, b, (dims, ((), ())),
                               preferred_element_type=jnp.float32)


def _rvq_body(h_ref, hi_ref, mid_ref, lo_ref, codes_ref, quant_ref,
              r_ref, cn_ref):
    jblk = pl.program_id(0)
    lvl = pl.program_id(1)

    c_hi = hi_ref[0]
    c_mid = mid_ref[0]
    c_lo = lo_ref[0]

    @pl.when(jblk == 0)
    def _():
        cb_f32 = (c_hi.astype(jnp.float32) + c_mid.astype(jnp.float32)
                  ) + c_lo.astype(jnp.float32)
        cbsq = cb_f32 * cb_f32
        sq_hi = cbsq.astype(jnp.bfloat16)
        sq_lo = (cbsq - sq_hi.astype(jnp.float32)).astype(jnp.bfloat16)
        ones_row = jnp.ones((1, DIM), jnp.bfloat16)
        cn_ref[lvl] = (_dot(ones_row, sq_hi, ((1,), (1,)))
                       + _dot(ones_row, sq_lo, ((1,), (1,))))

    cnorm = cn_ref[lvl]

    @pl.when(lvl == 0)
    def _():
        r_ref[...] = h_ref[...]
        quant_ref[...] = jnp.zeros_like(quant_ref)

    r = r_ref[...]
    r_hi = r.astype(jnp.bfloat16)
    r_lo = (r - r_hi.astype(jnp.float32)).astype(jnp.bfloat16)
    lane = jax.lax.broadcasted_iota(jnp.int32, (r.shape[0], BINS), 1)
    rc = (_dot(r_hi, c_hi, ((1,), (1,)))
          + _dot(r_hi, c_mid, ((1,), (1,)))
          + _dot(r_lo, c_hi, ((1,), (1,))))
    scores = cnorm - 2.0 * rc
    m1 = jnp.min(scores, axis=1, keepdims=True)
    i1 = jnp.min(jnp.where(scores == m1, lane, BINS), axis=1, keepdims=True)
    masked = jnp.where(lane == i1, jnp.inf, scores)
    m2 = jnp.min(masked, axis=1, keepdims=True)
    i2 = jnp.min(jnp.where(masked == m2, lane, BINS), axis=1, keepdims=True)
    t = r.shape[0]
    oh = jnp.concatenate([(lane == i1).astype(jnp.float32),
                          (lane == i2).astype(jnp.float32)],
                         axis=0).astype(jnp.bfloat16)
    c12 = (_dot(oh, c_hi, ((1,), (0,)))
           + _dot(oh, c_mid, ((1,), (0,)))
           + _dot(oh, c_lo, ((1,), (0,))))
    c1 = c12[:t]
    c2 = c12[t:]
    d1 = jnp.sum((r - c1) ** 2, axis=1, keepdims=True)
    d2 = jnp.sum((r - c2) ** 2, axis=1, keepdims=True)
    pick2 = (d2 < d1) | ((d2 == d1) & (i2 < i1))
    chosen = jnp.where(pick2, c2, c1)
    idx_col = jnp.where(pick2, i2, i1)
    digits = jnp.concatenate(
        [idx_col >> 8, idx_col & 255], axis=1).astype(jnp.bfloat16)
    eye = (jax.lax.broadcasted_iota(jnp.int32, (t, t), 0) ==
           jax.lax.broadcasted_iota(jnp.int32, (t, t), 1)).astype(jnp.bfloat16)
    ab_row = _dot(digits, eye, ((0,), (0,)))
    idx_row = ab_row[:1] * 256.0 + ab_row[1:]
    codes_ref[0] = idx_row.astype(jnp.int32)
    quant_ref[...] += chosen
    r_ref[...] = r - chosen


def kernel(hidden_states, codebooks):
    cb_hi = jax.lax.optimization_barrier(codebooks.astype(jnp.bfloat16))
    res1 = codebooks - cb_hi.astype(jnp.float32)
    cb_mid = jax.lax.optimization_barrier(res1.astype(jnp.bfloat16))
    cb_lo = (res1 - cb_mid.astype(jnp.float32)).astype(jnp.bfloat16)

    grid = (N_TOK // TOK_BLOCK, N_Q)
    codes3, quantized = pl.pallas_call(
        _rvq_body,
        grid=grid,
        in_specs=[
            pl.BlockSpec((TOK_BLOCK, DIM), lambda j, i: (j, 0)),
            pl.BlockSpec((1, BINS, DIM), lambda j, i: (i, 0, 0)),
            pl.BlockSpec((1, BINS, DIM), lambda j, i: (i, 0, 0)),
            pl.BlockSpec((1, BINS, DIM), lambda j, i: (i, 0, 0)),
        ],
        out_specs=[
            pl.BlockSpec((1, 1, TOK_BLOCK), lambda j, i: (i, 0, j)),
            pl.BlockSpec((TOK_BLOCK, DIM), lambda j, i: (j, 0)),
        ],
        out_shape=[
            jax.ShapeDtypeStruct((N_Q, 1, N_TOK), jnp.int32),
            jax.ShapeDtypeStruct((N_TOK, DIM), jnp.float32),
        ],
        scratch_shapes=[
            pltpu.VMEM((TOK_BLOCK, DIM), jnp.float32),
            pltpu.VMEM((N_Q, 1, BINS), jnp.float32),
        ],
    )(hidden_states, cb_hi, cb_mid, cb_lo)
    return codes3.reshape(N_Q, N_TOK), quantized


# speedup vs baseline: 1.4353x; 1.0059x over previous
import jax
import jax.numpy as jnp
from jax.experimental import pallas as pl
from jax.experimental.pallas import tpu as pltpu

DIM = 256
N_Q = 8
BINS = 1024
N_TOK = 2048
TOK_BLOCK = 2048


def _dot(a, b, dims):
    return jax.lax.dot_general(a, b, (dims, ((), ())),
                               preferred_element_type=jnp.float32)


def _rvq_body(h_ref, hi_ref, mid_ref, lo_ref, codes_ref, quant_ref,
              r_ref, cn_ref):
    jblk = pl.program_id(0)
    lvl = pl.program_id(1)

    c_hi = hi_ref[0]
    c_mid = mid_ref[0]
    c_lo = lo_ref[0]

    @pl.when(jblk == 0)
    def _():
        cb_f32 = (c_hi.astype(jnp.float32) + c_mid.astype(jnp.float32)
                  ) + c_lo.astype(jnp.float32)
        cbsq = cb_f32 * cb_f32
        sq_hi = cbsq.astype(jnp.bfloat16)
        sq_lo = (cbsq - sq_hi.astype(jnp.float32)).astype(jnp.bfloat16)
        ones_row = jnp.ones((1, DIM), jnp.bfloat16)
        cn_ref[lvl] = (_dot(ones_row, sq_hi, ((1,), (1,)))
                       + _dot(ones_row, sq_lo, ((1,), (1,))))

    cnorm = cn_ref[lvl]

    @pl.when(lvl == 0)
    def _():
        r_ref[...] = h_ref[...]
        quant_ref[...] = jnp.zeros_like(quant_ref)

    r = r_ref[...]
    r_hi = r.astype(jnp.bfloat16)
    r_lo = (r - r_hi.astype(jnp.float32)).astype(jnp.bfloat16)
    lane = jax.lax.broadcasted_iota(jnp.int32, (r.shape[0], BINS), 1)
    rc = (_dot(r_hi, c_hi, ((1,), (1,)))
          + _dot(r_hi, c_mid, ((1,), (1,)))
          + _dot(r_lo, c_hi, ((1,), (1,))))
    scores = cnorm - 2.0 * rc
    m1 = jnp.min(scores, axis=1, keepdims=True)
    i1 = jnp.min(jnp.where(scores == m1, lane, BINS), axis=1, keepdims=True)
    masked = jnp.where(lane == i1, jnp.inf, scores)
    m2 = jnp.min(masked, axis=1, keepdims=True)
    i2 = jnp.min(jnp.where(masked == m2, lane, BINS), axis=1, keepdims=True)
    t = r.shape[0]
    oh = jnp.concatenate([(lane == i1).astype(jnp.float32),
                          (lane == i2).astype(jnp.float32)],
                         axis=0).astype(jnp.bfloat16)
    c12 = (_dot(oh, c_hi, ((1,), (0,)))
           + _dot(oh, c_mid, ((1,), (0,)))
           + _dot(oh, c_lo, ((1,), (0,))))
    c1 = c12[:t]
    c2 = c12[t:]
    d1 = jnp.sum((r - c1) ** 2, axis=1, keepdims=True)
    d2 = jnp.sum((r - c2) ** 2, axis=1, keepdims=True)
    pick2 = (d2 < d1) | ((d2 == d1) & (i2 < i1))
    chosen = jnp.where(pick2, c2, c1)
    idx_col = jnp.where(pick2, i2, i1)
    digits = jnp.concatenate(
        [idx_col >> 8, idx_col & 255], axis=1).astype(jnp.bfloat16)
    eye = (jax.lax.broadcasted_iota(jnp.int32, (t, t), 0) ==
           jax.lax.broadcasted_iota(jnp.int32, (t, t), 1)).astype(jnp.bfloat16)
    ab_row = _dot(digits, eye, ((0,), (0,)))
    idx_row = ab_row[:1] * 256.0 + ab_row[1:]
    codes_ref[0] = idx_row.astype(jnp.int32)
    quant_ref[...] += chosen
    r_ref[...] = r - chosen


def kernel(hidden_states, codebooks):
    cb_hi = jax.lax.optimization_barrier(codebooks.astype(jnp.bfloat16))
    res1 = codebooks - cb_hi.astype(jnp.float32)
    cb_mid = jax.lax.optimization_barrier(res1.astype(jnp.bfloat16))
    cb_lo = (res1 - cb_mid.astype(jnp.float32)).astype(jnp.bfloat16)

    grid = (N_TOK // TOK_BLOCK, N_Q)
    codes3, quantized = pl.pallas_call(
        _rvq_body,
        grid=grid,
        in_specs=[
            pl.BlockSpec((TOK_BLOCK, DIM), lambda j, i: (j, 0)),
            pl.BlockSpec((1, BINS, DIM), lambda j, i: (i, 0, 0)),
            pl.BlockSpec((1, BINS, DIM), lambda j, i: (i, 0, 0)),
            pl.BlockSpec((1, BINS, DIM), lambda j, i: (i, 0, 0)),
        ],
        out_specs=[
            pl.BlockSpec((1, 1, TOK_BLOCK), lambda j, i: (i, 0, j)),
            pl.BlockSpec((TOK_BLOCK, DIM), lambda j, i: (j, 0)),
        ],
        out_shape=[
            jax.ShapeDtypeStruct((N_Q, 1, N_TOK), jnp.int32),
            jax.ShapeDtypeStruct((N_TOK, DIM), jnp.float32),
        ],
        scratch_shapes=[
            pltpu.VMEM((TOK_BLOCK, DIM), jnp.float32),
            pltpu.VMEM((N_Q, 1, BINS), jnp.float32),
        ],
    )(hidden_states, cb_hi, cb_mid, cb_lo)
    return codes3.reshape(N_Q, N_TOK), quantized
